# Initial kernel scaffold; baseline (speedup 1.0000x reference)
import functools

import jax
import jax.numpy as jnp
from jax import lax
from jax.experimental import pallas as pl
from jax.experimental.pallas import tpu as pltpu
from jax.experimental.pallas import tpu_sc as plsc

NC = 2
NS = 16
NW = NC * NS
GH = 64
GW = 64
BATCH = 256
VD = 64
PD = 64
ED = VD + PD
PLANE = GH * GW
TOTAL = BATCH * PLANE
PER_W = TOTAL // NW
CHUNK = 512
NCHUNK = PER_W // CHUNK
GSUB = 128
KSUB = CHUNK // GSUB
PLANES_PER_W = PER_W // PLANE


def _sc_body(idx_hbm, vtab_hbm, rowp_hbm, colp_hbm, out_hbm,
             idx_v, val_v, rowp_v, colp_v, pbuf_v, pos_sh, sem):
    cid = lax.axis_index("c")
    sid = lax.axis_index("s")
    wid = sid * NC + cid

    pltpu.sync_copy(rowp_hbm, rowp_v)
    pltpu.sync_copy(colp_hbm, colp_v)

    for hh in range(GH // NS):
        h = sid * (GH // NS) + hh

        def w_body(w, _):
            pbuf_v[w, pl.ds(0, 16)] = rowp_v[h, pl.ds(0, 16)]
            pbuf_v[w, pl.ds(16, 16)] = rowp_v[h, pl.ds(16, 16)]
            pbuf_v[w, pl.ds(32, 16)] = colp_v[w, pl.ds(0, 16)]
            pbuf_v[w, pl.ds(48, 16)] = colp_v[w, pl.ds(16, 16)]
            return 0

        lax.fori_loop(0, GW, w_body, 0)
        pltpu.sync_copy(pbuf_v, pos_sh.at[pl.ds(h * GW, GW)])
    plsc.subcore_barrier()

    base0 = wid * PER_W

    for p in range(PLANES_PER_W):
        pltpu.sync_copy(
            pos_sh,
            out_hbm.at[pl.ds(base0 + p * PLANE, PLANE), pl.ds(VD, PD)])

    def chunk_body(g, _):
        cbase = base0 + g * CHUNK
        pltpu.sync_copy(idx_hbm.at[pl.ds(cbase // GSUB, KSUB)], idx_v)
        cps = [
            pltpu.async_copy(
                vtab_hbm.at[idx_v.at[j]],
                val_v.at[pl.ds(j * GSUB, GSUB)],
                sem)
            for j in range(KSUB)
        ]
        for cp in cps:
            cp.wait()
        pltpu.sync_copy(val_v, out_hbm.at[pl.ds(cbase, CHUNK), pl.ds(0, VD)])
        return 0

    lax.fori_loop(0, NCHUNK, chunk_body, 0)


_mesh = plsc.VectorSubcoreMesh(
    core_axis_name="c", subcore_axis_name="s", num_cores=NC, num_subcores=NS)

_sc_call = functools.partial(
    pl.kernel,
    out_type=jax.ShapeDtypeStruct((TOTAL, ED), jnp.float32),
    mesh=_mesh,
    scratch_types=[
        pltpu.VMEM((KSUB, GSUB), jnp.int32),
        pltpu.VMEM((CHUNK, VD), jnp.float32),
        pltpu.VMEM((GH, 32), jnp.float32),
        pltpu.VMEM((GW, 32), jnp.float32),
        pltpu.VMEM((GW, PD), jnp.float32),
        pltpu.VMEM_SHARED((PLANE, PD), jnp.float32),
        pltpu.SemaphoreType.DMA,
    ],
)(_sc_body)


@jax.jit
def kernel(grid, value_embed, row_pos_embed, col_pos_embed):
    idx = grid.astype(jnp.int32).reshape(TOTAL // GSUB, GSUB)
    out = _sc_call(idx, value_embed, row_pos_embed, col_pos_embed)
    return out.reshape(BATCH, GH, GW, ED)


# speedup vs baseline: 6.7347x; 6.7347x over previous
import functools

import jax
import jax.numpy as jnp
from jax import lax
from jax.experimental import pallas as pl
from jax.experimental.pallas import tpu as pltpu
from jax.experimental.pallas import tpu_sc as plsc

NC = 2
NS = 16
NW = NC * NS
GH = 64
GW = 64
BATCH = 256
VD = 64
PD = 64
ED = VD + PD
PLANE = GH * GW
TOTAL = BATCH * PLANE
PER_W = TOTAL // NW
CHUNK = 512
NCHUNK = PER_W // CHUNK
GSUB = 128
KSUB = CHUNK // GSUB
PLANES_PER_W = PER_W // PLANE


def _sc_body(idx_hbm, vtab_hbm, rowp_hbm, colp_hbm, out_hbm,
             idx_v, val_v, rowp_v, colp_v, pbuf_v, pos_sh, sem):
    cid = lax.axis_index("c")
    sid = lax.axis_index("s")
    wid = sid * NC + cid

    pltpu.sync_copy(rowp_hbm, rowp_v)
    pltpu.sync_copy(colp_hbm, colp_v)

    for hh in range(GH // NS):
        h = sid * (GH // NS) + hh

        def w_body(w, _):
            pbuf_v[w, pl.ds(0, 16)] = rowp_v[h, pl.ds(0, 16)]
            pbuf_v[w, pl.ds(16, 16)] = rowp_v[h, pl.ds(16, 16)]
            pbuf_v[w, pl.ds(32, 16)] = colp_v[w, pl.ds(0, 16)]
            pbuf_v[w, pl.ds(48, 16)] = colp_v[w, pl.ds(16, 16)]
            return 0

        lax.fori_loop(0, GW, w_body, 0)
        pltpu.sync_copy(pbuf_v, pos_sh.at[pl.ds(h * GW, GW)])
    plsc.subcore_barrier()

    base0 = wid * PER_W

    for p in range(PLANES_PER_W):
        pltpu.sync_copy(
            pos_sh,
            out_hbm.at[pl.ds(base0 + p * PLANE, PLANE), pl.ds(VD, PD)])

    def chunk_body(g, _):
        cbase = base0 + g * CHUNK
        pltpu.sync_copy(idx_hbm.at[pl.ds(cbase // GSUB, KSUB)], idx_v)
        cps = [
            pltpu.async_copy(
                vtab_hbm.at[idx_v.at[j]],
                val_v.at[pl.ds(j * GSUB, GSUB)],
                sem)
            for j in range(KSUB)
        ]
        for cp in cps:
            cp.wait()
        pltpu.sync_copy(val_v, out_hbm.at[pl.ds(cbase, CHUNK), pl.ds(0, VD)])
        return 0

    lax.fori_loop(0, NCHUNK, chunk_body, 0)


_mesh = plsc.VectorSubcoreMesh(
    core_axis_name="c", subcore_axis_name="s", num_cores=NC, num_subcores=NS)

_sc_call = functools.partial(
    pl.kernel,
    out_type=jax.ShapeDtypeStruct((TOTAL, ED), jnp.float32),
    mesh=_mesh,
    scratch_types=[
        pltpu.VMEM((KSUB, GSUB), jnp.int32),
        pltpu.VMEM((CHUNK, VD), jnp.float32),
        pltpu.VMEM((GH, 32), jnp.float32),
        pltpu.VMEM((GW, 32), jnp.float32),
        pltpu.VMEM((GW, PD), jnp.float32),
        pltpu.VMEM_SHARED((PLANE, PD), jnp.float32),
        pltpu.SemaphoreType.DMA,
    ],
    compiler_params=pltpu.CompilerParams(use_tc_tiling_on_sc=False),
)(_sc_body)


@jax.jit
def kernel(grid, value_embed, row_pos_embed, col_pos_embed):
    idx = grid.astype(jnp.int32).reshape(TOTAL // GSUB, GSUB)
    out = _sc_call(idx, value_embed, row_pos_embed, col_pos_embed)
    return out.reshape(BATCH, GH, GW, ED)


# speedup vs baseline: 8.1577x; 1.2113x over previous
import functools

import jax
import jax.numpy as jnp
from jax import lax
from jax.experimental import pallas as pl
from jax.experimental.pallas import tpu as pltpu
from jax.experimental.pallas import tpu_sc as plsc

NC = 2
NS = 16
NW = NC * NS
GH = 64
GW = 64
BATCH = 256
VD = 64
PD = 64
ED = VD + PD
PLANE = GH * GW
TOTAL = BATCH * PLANE
PER_W = TOTAL // NW
CHUNK = 512
NCHUNK = PER_W // CHUNK
GSUB = 128
KSUB = CHUNK // GSUB
PLANES_PER_W = PER_W // PLANE


def _sc_body(idx_hbm, vtab_hbm, rowp_hbm, colp_hbm, out_hbm,
             idx_v, val0_v, val1_v, rowp_v, colp_v, pbuf_v, pos_sh,
             gsem, wsem0, wsem1, psem):
    cid = lax.axis_index("c")
    sid = lax.axis_index("s")
    wid = sid * NC + cid

    pltpu.sync_copy(rowp_hbm, rowp_v)
    pltpu.sync_copy(colp_hbm, colp_v)

    for hh in range(GH // NS):
        h = sid * (GH // NS) + hh

        def w_body(w, _):
            pbuf_v[w, pl.ds(0, 16)] = rowp_v[h, pl.ds(0, 16)]
            pbuf_v[w, pl.ds(16, 16)] = rowp_v[h, pl.ds(16, 16)]
            pbuf_v[w, pl.ds(32, 16)] = colp_v[w, pl.ds(0, 16)]
            pbuf_v[w, pl.ds(48, 16)] = colp_v[w, pl.ds(16, 16)]
            return 0

        lax.fori_loop(0, GW, w_body, 0)
        pltpu.sync_copy(pbuf_v, pos_sh.at[pl.ds(h * GW, GW)])
    plsc.subcore_barrier()

    base0 = wid * PER_W

    def pos_dst(p):
        return out_hbm.at[pl.ds(base0 + p * PLANE, PLANE), pl.ds(VD, PD)]

    for p in range(PLANES_PER_W):
        pltpu.async_copy(pos_sh, pos_dst(p), psem)

    def val_dst(cbase):
        return out_hbm.at[pl.ds(cbase, CHUNK), pl.ds(0, VD)]

    def pair_body(t, _):
        for b, val_v, wsem in ((0, val0_v, wsem0), (1, val1_v, wsem1)):
            g = 2 * t + b
            cbase = base0 + g * CHUNK
            pltpu.sync_copy(
                idx_hbm.at[pl.ds(base0 // GSUB + g * KSUB, KSUB)], idx_v)

            @pl.when(t > 0)
            def _wait_prev_write():
                pltpu.make_async_copy(val_v, val_dst(cbase), wsem).wait()

            cps = [
                pltpu.async_copy(
                    vtab_hbm.at[idx_v.at[j]],
                    val_v.at[pl.ds(j * GSUB, GSUB)],
                    gsem)
                for j in range(KSUB)
            ]
            for cp in cps:
                cp.wait()
            pltpu.async_copy(val_v, val_dst(cbase), wsem)
        return 0

    lax.fori_loop(0, NCHUNK // 2, pair_body, 0)

    pltpu.make_async_copy(val0_v, val_dst(base0), wsem0).wait()
    pltpu.make_async_copy(val1_v, val_dst(base0), wsem1).wait()
    for p in range(PLANES_PER_W):
        pltpu.make_async_copy(pos_sh, pos_dst(p), psem).wait()


_mesh = plsc.VectorSubcoreMesh(
    core_axis_name="c", subcore_axis_name="s", num_cores=NC, num_subcores=NS)

_sc_call = functools.partial(
    pl.kernel,
    out_type=jax.ShapeDtypeStruct((TOTAL, ED), jnp.float32),
    mesh=_mesh,
    scratch_types=[
        pltpu.VMEM((KSUB, GSUB), jnp.int32),
        pltpu.VMEM((CHUNK, VD), jnp.float32),
        pltpu.VMEM((CHUNK, VD), jnp.float32),
        pltpu.VMEM((GH, 32), jnp.float32),
        pltpu.VMEM((GW, 32), jnp.float32),
        pltpu.VMEM((GW, PD), jnp.float32),
        pltpu.VMEM_SHARED((PLANE, PD), jnp.float32),
        pltpu.SemaphoreType.DMA,
        pltpu.SemaphoreType.DMA,
        pltpu.SemaphoreType.DMA,
        pltpu.SemaphoreType.DMA,
    ],
    compiler_params=pltpu.CompilerParams(use_tc_tiling_on_sc=False),
)(_sc_body)


@jax.jit
def kernel(grid, value_embed, row_pos_embed, col_pos_embed):
    idx = grid.astype(jnp.int32).reshape(TOTAL // GSUB, GSUB)
    out = _sc_call(idx, value_embed, row_pos_embed, col_pos_embed)
    return out.reshape(BATCH, GH, GW, ED)


# speedup vs baseline: 17.1751x; 2.1054x over previous
import functools

import jax
import jax.numpy as jnp
from jax import lax
from jax.experimental import pallas as pl
from jax.experimental.pallas import tpu as pltpu
from jax.experimental.pallas import tpu_sc as plsc

NC = 2
NS = 16
NW = NC * NS
GH = 64
GW = 64
BATCH = 256
VD = 64
PD = 64
ED = VD + PD
PLANE = GH * GW
TOTAL = BATCH * PLANE
PER_W = TOTAL // NW
CHUNK = 512
NCHUNK = PER_W // CHUNK
GSUB = 128
KSUB = CHUNK // GSUB
PLANES_PER_W = PER_W // PLANE


def _sc_body(idx_hbm, vtab_hbm, rowp_hbm, colp_hbm, out_hbm,
             idx_v, val0_v, val1_v, rowp_v, colp_v, pbuf_v, pos_sh, tab_sh,
             gsem, wsem0, wsem1, psem):
    cid = lax.axis_index("c")
    sid = lax.axis_index("s")
    wid = sid * NC + cid

    @pl.when(sid == 0)
    def _stage_table():
        pltpu.sync_copy(vtab_hbm, tab_sh)

    pltpu.sync_copy(rowp_hbm, rowp_v)
    pltpu.sync_copy(colp_hbm, colp_v)

    for hh in range(GH // NS):
        h = sid * (GH // NS) + hh

        def w_body(w, _):
            pbuf_v[w, pl.ds(0, 16)] = rowp_v[h, pl.ds(0, 16)]
            pbuf_v[w, pl.ds(16, 16)] = rowp_v[h, pl.ds(16, 16)]
            pbuf_v[w, pl.ds(32, 16)] = colp_v[w, pl.ds(0, 16)]
            pbuf_v[w, pl.ds(48, 16)] = colp_v[w, pl.ds(16, 16)]
            return 0

        lax.fori_loop(0, GW, w_body, 0)
        pltpu.sync_copy(pbuf_v, pos_sh.at[pl.ds(h * GW, GW)])
    plsc.subcore_barrier()

    base0 = wid * PER_W

    def pos_dst(p):
        return out_hbm.at[pl.ds(base0 + p * PLANE, PLANE), pl.ds(VD, PD)]

    for p in range(PLANES_PER_W):
        pltpu.async_copy(pos_sh, pos_dst(p), psem)

    def val_dst(cbase):
        return out_hbm.at[pl.ds(cbase, CHUNK), pl.ds(0, VD)]

    def pair_body(t, _):
        for b, val_v, wsem in ((0, val0_v, wsem0), (1, val1_v, wsem1)):
            g = 2 * t + b
            cbase = base0 + g * CHUNK
            pltpu.sync_copy(
                idx_hbm.at[pl.ds(base0 // GSUB + g * KSUB, KSUB)], idx_v)

            @pl.when(t > 0)
            def _wait_prev_write():
                pltpu.make_async_copy(val_v, val_dst(cbase), wsem).wait()

            cps = [
                pltpu.async_copy(
                    tab_sh.at[idx_v.at[j]],
                    val_v.at[pl.ds(j * GSUB, GSUB)],
                    gsem)
                for j in range(KSUB)
            ]
            for cp in cps:
                cp.wait()
            pltpu.async_copy(val_v, val_dst(cbase), wsem)
        return 0

    lax.fori_loop(0, NCHUNK // 2, pair_body, 0)

    pltpu.make_async_copy(val0_v, val_dst(base0), wsem0).wait()
    pltpu.make_async_copy(val1_v, val_dst(base0), wsem1).wait()
    for p in range(PLANES_PER_W):
        pltpu.make_async_copy(pos_sh, pos_dst(p), psem).wait()


_mesh = plsc.VectorSubcoreMesh(
    core_axis_name="c", subcore_axis_name="s", num_cores=NC, num_subcores=NS)

_sc_call = functools.partial(
    pl.kernel,
    out_type=jax.ShapeDtypeStruct((TOTAL, ED), jnp.float32),
    mesh=_mesh,
    scratch_types=[
        pltpu.VMEM((KSUB, GSUB), jnp.int32),
        pltpu.VMEM((CHUNK, VD), jnp.float32),
        pltpu.VMEM((CHUNK, VD), jnp.float32),
        pltpu.VMEM((GH, 32), jnp.float32),
        pltpu.VMEM((GW, 32), jnp.float32),
        pltpu.VMEM((GW, PD), jnp.float32),
        pltpu.VMEM_SHARED((PLANE, PD), jnp.float32),
        pltpu.VMEM_SHARED((1024, VD), jnp.float32),
        pltpu.SemaphoreType.DMA,
        pltpu.SemaphoreType.DMA,
        pltpu.SemaphoreType.DMA,
        pltpu.SemaphoreType.DMA,
    ],
    compiler_params=pltpu.CompilerParams(use_tc_tiling_on_sc=False),
)(_sc_body)


@jax.jit
def kernel(grid, value_embed, row_pos_embed, col_pos_embed):
    idx = grid.astype(jnp.int32).reshape(TOTAL // GSUB, GSUB)
    out = _sc_call(idx, value_embed, row_pos_embed, col_pos_embed)
    return out.reshape(BATCH, GH, GW, ED)
